# baseline (speedup 1.0000x reference)
import functools

import jax
import jax.numpy as jnp
from jax import lax
from jax.experimental import pallas as pl
from jax.experimental.pallas import tpu as pltpu
from jax.experimental.pallas import tpu_sc as plsc

NC = 2
NS = 16
NW = NC * NS
LANES = 16
IDX_CHUNK = 128


@functools.partial(jax.jit, static_argnums=(3, 4))
def _sc_lookup(idx2, weight_mu, weight_logvar, b_per_w, n_chunks):
    B = idx2.shape[0] * idx2.shape[1]
    D = weight_mu.shape[1]
    rows_per_w = b_per_w // IDX_CHUNK
    mesh = plsc.VectorSubcoreMesh(
        core_axis_name="c", subcore_axis_name="s",
        num_cores=NC, num_subcores=NS)

    @functools.partial(
        pl.kernel,
        out_type=[
            jax.ShapeDtypeStruct((B, D), jnp.float32),
            jax.ShapeDtypeStruct((B, D), jnp.float32),
            jax.ShapeDtypeStruct((B, D), jnp.float32),
        ],
        mesh=mesh,
        compiler_params=pltpu.CompilerParams(use_tc_tiling_on_sc=False),
        scratch_types=[
            pltpu.VMEM((rows_per_w, IDX_CHUNK), jnp.int32),
            pltpu.VMEM((b_per_w, D), jnp.float32),
            pltpu.VMEM((b_per_w, D), jnp.float32),
            pltpu.VMEM((b_per_w, D), jnp.float32),
            pltpu.SemaphoreType.DMA,
        ],
    )
    def k(idx_hbm, mu_hbm, lv_hbm, out_mu, out_lv, out_std,
          idx_v, rows_mu, rows_lv, rows_std, sem):
        wid = lax.axis_index("s") * NC + lax.axis_index("c")
        base = wid * b_per_w

        pltpu.sync_copy(idx_hbm.at[pl.ds(wid * rows_per_w, rows_per_w)], idx_v)

        copies = []
        for j in range(n_chunks):
            off = j * IDX_CHUNK
            copies.append(pltpu.async_copy(
                mu_hbm.at[idx_v.at[j]], rows_mu.at[pl.ds(off, IDX_CHUNK)], sem))
            copies.append(pltpu.async_copy(
                lv_hbm.at[idx_v.at[j]], rows_lv.at[pl.ds(off, IDX_CHUNK)], sem))
        for c in copies:
            c.wait()

        n_vec = D // LANES

        def body(i, _):
            r0 = i * 4
            for kk in range(4):
                for j in range(n_vec):
                    v = rows_lv[r0 + kk, pl.ds(j * LANES, LANES)]
                    rows_std[r0 + kk, pl.ds(j * LANES, LANES)] = jnp.exp(0.5 * v)
            return 0

        lax.fori_loop(0, b_per_w // 4, body, 0)

        pltpu.sync_copy(rows_mu, out_mu.at[pl.ds(base, b_per_w)])
        pltpu.sync_copy(rows_lv, out_lv.at[pl.ds(base, b_per_w)])
        pltpu.sync_copy(rows_std, out_std.at[pl.ds(base, b_per_w)])

    return k(idx2, weight_mu, weight_logvar)


def kernel(idx, num_augment_pts, weight_mu, weight_logvar):
    del num_augment_pts
    B = idx.shape[0]
    assert B % (NW * IDX_CHUNK) == 0
    b_per_w = B // NW
    n_chunks = b_per_w // IDX_CHUNK
    idx2 = idx.astype(jnp.int32).reshape(B // IDX_CHUNK, IDX_CHUNK)
    mu, logvar, std = _sc_lookup(idx2, weight_mu, weight_logvar,
                                 b_per_w, n_chunks)
    return (mu, mu, logvar, std)
